# baseline (speedup 1.0000x reference)
import functools

import jax
import jax.numpy as jnp
from jax import lax
from jax.experimental import pallas as pl
from jax.experimental.pallas import tpu as pltpu
from jax.experimental.pallas import tpu_sc as plsc

N = 10000
D = 128
DH = D // 2
NE = 320000
NPAD = 10240
CHK = 128
NCHNK = 160
EPT = NCHNK * CHK
NE_PAD = EPT * 16
NROW2 = NE_PAD // CHK
G = 16
DCHK = 128
DEG_ROWS = NE_PAD // DCHK // 32
ROWS_PER_TILE = NPAD // 16

_MESH = plsc.VectorSubcoreMesh(core_axis_name="c", subcore_axis_name="s")


@functools.partial(
    pl.kernel,
    out_type=jax.ShapeDtypeStruct((2, NPAD), jnp.float32),
    mesh=_MESH,
    scratch_types=[
        pltpu.VMEM((DEG_ROWS, DCHK), jnp.int32),
        pltpu.VMEM((DCHK,), jnp.float32),
        pltpu.VMEM((NPAD // 16,), jnp.float32),
        pltpu.VMEM_SHARED((NPAD,), jnp.float32),
    ],
)
def _deg_kernel(dstd_hbm, out_hbm, di_v, ones_v, z_v, acc):
    c = lax.axis_index("c")
    s = lax.axis_index("s")
    for i in range(DCHK // 16):
        ones_v[pl.ds(i * 16, 16)] = jnp.full((16,), 1.0, jnp.float32)
    for i in range((NPAD // 16) // 16):
        z_v[pl.ds(i * 16, 16)] = jnp.zeros((16,), jnp.float32)
    strip = NPAD // 16
    pltpu.sync_copy(z_v, acc.at[pl.ds(s * strip, strip)])
    w = c * 16 + s
    pltpu.sync_copy(dstd_hbm.at[pl.ds(w * DEG_ROWS, DEG_ROWS)], di_v)
    plsc.subcore_barrier()

    def body(i, _):
        pltpu.sync_copy(ones_v, acc.at[di_v.at[i]], add=True)
        return ()

    lax.fori_loop(0, DEG_ROWS, body, ())
    plsc.subcore_barrier()
    pltpu.sync_copy(acc.at[pl.ds(s * strip, strip)],
                    out_hbm.at[c, pl.ds(s * strip, strip)])


@functools.partial(
    pl.kernel,
    out_type=jax.ShapeDtypeStruct((2 * NPAD, D), jnp.float32),
    mesh=_MESH,
    scratch_types=[
        pltpu.VMEM((G, CHK), jnp.int32),
        pltpu.VMEM((G, CHK), jnp.int32),
        pltpu.VMEM((CHK, D), jnp.float32),
        pltpu.VMEM_SHARED((NPAD, D), jnp.float32),
    ],
)
def _prop_kernel(hs_hbm, src2p_hbm, dst2_hbm, zrows_hbm, out_hbm,
                 si_v, di_v, rows, acc):
    c = lax.axis_index("c")
    s = lax.axis_index("s")
    row0 = s * ROWS_PER_TILE

    def group(g, _):
        pltpu.sync_copy(
            src2p_hbm.at[pl.ds(c * NROW2 + s * NCHNK + g * G, G)], si_v)
        pltpu.sync_copy(dst2_hbm.at[pl.ds(s * NCHNK + g * G, G)], di_v)

        def chunk(j, _):
            pltpu.sync_copy(hs_hbm.at[si_v.at[j]], rows)
            pltpu.sync_copy(rows, acc.at[di_v.at[j]], add=True)
            return ()

        lax.fori_loop(0, G, chunk, ())
        return ()

    for q in range(ROWS_PER_TILE // CHK):
        pltpu.sync_copy(zrows_hbm, acc.at[pl.ds(row0 + q * CHK, CHK)])
    plsc.subcore_barrier()
    lax.fori_loop(0, NCHNK // G, group, ())
    plsc.subcore_barrier()
    pltpu.sync_copy(acc.at[pl.ds(row0, ROWS_PER_TILE)],
                    out_hbm.at[pl.ds(c * NPAD + row0, ROWS_PER_TILE)])


_RB = 2000


def _mm_body(x_ref, w_ref, b_ref, degt_ref, hhp_ref, hs_ref, hi_ref):
    xb = x_ref[...]
    acc = jnp.dot(xb, w_ref[...], preferred_element_type=jnp.float32)
    acc = acc + b_ref[0:1, :]
    degb = degt_ref[...]
    deg = degb[:, 0:1] + degb[:, 1:2]
    dinv = jnp.where(deg > 0, lax.rsqrt(deg), 0.0)
    h_hp = acc[:, 0:D]
    h_lp = acc[:, D:2 * D]
    h_i = acc[:, 2 * D:3 * D]
    hhp_ref[...] = h_hp
    hs_ref[0] = dinv * h_hp
    hs_ref[1] = dinv * h_lp
    hi_ref[...] = jnp.maximum(h_i, 0.0)


def _mm_call(x, w_cat, b_cat, deg_t):
    return pl.pallas_call(
        _mm_body,
        grid=(N // _RB,),
        in_specs=[
            pl.BlockSpec((_RB, D), lambda i: (i, 0)),
            pl.BlockSpec((D, 3 * D), lambda i: (0, 0)),
            pl.BlockSpec((8, 3 * D), lambda i: (0, 0)),
            pl.BlockSpec((_RB, 2), lambda i: (i, 0)),
        ],
        out_specs=[
            pl.BlockSpec((_RB, D), lambda i: (i, 0)),
            pl.BlockSpec((2, _RB, D), lambda i: (0, i, 0)),
            pl.BlockSpec((_RB, D), lambda i: (i, 0)),
        ],
        out_shape=[
            jax.ShapeDtypeStruct((N, D), jnp.float32),
            jax.ShapeDtypeStruct((2, NPAD, D), jnp.float32),
            jax.ShapeDtypeStruct((N, D), jnp.float32),
        ],
    )(x, w_cat, b_cat, deg_t)


def _fin_body(hhp_ref, prop_ref, hi_ref, degt_ref, gw_ref,
              hhp0_ref, prop0_ref, hi0_ref, deg0_ref, out_ref):
    d0 = deg0_ref[0, 0] + deg0_ref[0, 1]
    dinv0 = jnp.where(d0 > 0, lax.rsqrt(d0), 0.0)
    php0 = prop0_ref[0, 0:1, :]
    plp0 = prop0_ref[1, 0:1, :]
    hhp0 = jnp.maximum(hhp0_ref[0:1, :] - dinv0 * php0, 0.0)
    hlp0 = jnp.maximum(dinv0 * plp0, 0.0)
    hi0 = hi0_ref[0:1, :]
    ga = jnp.sum(hhp0 * gw_ref[0:1, :]) + gw_ref[3, 0]
    gb = jnp.sum(hlp0 * gw_ref[1:2, :]) + gw_ref[3, 1]
    gc = jnp.sum(hi0 * gw_ref[2:3, :]) + gw_ref[3, 2]

    degb = degt_ref[...]
    deg = degb[:, 0:1] + degb[:, 1:2]
    dinv = jnp.where(deg > 0, lax.rsqrt(deg), 0.0)
    php = prop_ref[0]
    plp = prop_ref[1]
    h_hp = jnp.maximum(hhp_ref[...] - dinv * php, 0.0)
    h_lp = jnp.maximum(dinv * plp, 0.0)
    z = ga * h_hp + gb * h_lp + gc * hi_ref[...]
    m = jnp.max(z, axis=1, keepdims=True)
    zs = z - m
    out_ref[...] = zs - jnp.log(jnp.sum(jnp.exp(zs), axis=1, keepdims=True))


def _fin_call(h_hp, prop, h_i, deg_t, gw):
    return pl.pallas_call(
        _fin_body,
        grid=(N // _RB,),
        in_specs=[
            pl.BlockSpec((_RB, D), lambda i: (i, 0)),
            pl.BlockSpec((2, _RB, D), lambda i: (0, i, 0)),
            pl.BlockSpec((_RB, D), lambda i: (i, 0)),
            pl.BlockSpec((_RB, 2), lambda i: (i, 0)),
            pl.BlockSpec((8, D), lambda i: (0, 0)),
            pl.BlockSpec((8, D), lambda i: (0, 0)),
            pl.BlockSpec((2, 8, D), lambda i: (0, 0, 0)),
            pl.BlockSpec((8, D), lambda i: (0, 0)),
            pl.BlockSpec((8, 2), lambda i: (0, 0)),
        ],
        out_specs=pl.BlockSpec((_RB, D), lambda i: (i, 0)),
        out_shape=jax.ShapeDtypeStruct((N, D), jnp.float32),
    )(h_hp, prop, h_i, deg_t, gw, h_hp, prop, h_i, deg_t)


def kernel(x, edge_index, W_hp, b_hp, W_lp, b_lp, W_i, b_i,
           w_h, bh, w_l, bl, w_i, bi):
    src = edge_index[0]
    dst = edge_index[1]

    pad = NE_PAD - NE
    src_pad = jnp.concatenate([src, jnp.zeros((pad,), jnp.int32)])
    dst_pad = jnp.concatenate([dst, jnp.full((pad,), N, jnp.int32)])
    src2 = src_pad.reshape(NROW2, CHK)
    dst2 = dst_pad.reshape(NROW2, CHK)

    deg2 = _deg_kernel(dst2)
    deg_t = jnp.transpose(deg2)[:N]

    w_cat = jnp.concatenate([W_hp, W_lp, W_i], axis=1)
    b_cat = jnp.zeros((8, 3 * D), jnp.float32).at[0].set(
        jnp.concatenate([b_hp, b_lp, b_i]))

    h_hp, hs, h_i = _mm_call(x, w_cat, b_cat, deg_t)

    zrows = jnp.zeros((CHK, D), jnp.float32)

    hs_flat = hs.reshape(2 * NPAD, D)
    src2p = (src2[None, :, :]
             + (jnp.arange(2, dtype=jnp.int32) * NPAD)[:, None, None]
             ).reshape(2 * NROW2, CHK)

    prop = _prop_kernel(hs_flat, src2p, dst2, zrows).reshape(2, NPAD, D)

    gw = (jnp.zeros((8, D), jnp.float32)
          .at[0].set(w_h[:, 0]).at[1].set(w_l[:, 0]).at[2].set(w_i[:, 0])
          .at[3, 0].set(bh[0]).at[3, 1].set(bl[0]).at[3, 2].set(bi[0]))

    return _fin_call(h_hp, prop, h_i, deg_t, gw)


# speedup vs baseline: 1.0461x; 1.0461x over previous
import functools

import jax
import jax.numpy as jnp
from jax import lax
from jax.experimental import pallas as pl
from jax.experimental.pallas import tpu as pltpu
from jax.experimental.pallas import tpu_sc as plsc

N = 10000
D = 128
DH = D // 2
NE = 320000
NPAD = 10240
CHK = 128
NCHNK = 160
EPT = NCHNK * CHK
NE_PAD = EPT * 16
NROW2 = NE_PAD // CHK
G = 16
DCHK = 128
DEG_ROWS = NE_PAD // DCHK // 32
ROWS_PER_TILE = NPAD // 16

_MESH = plsc.VectorSubcoreMesh(core_axis_name="c", subcore_axis_name="s")


@functools.partial(
    pl.kernel,
    out_type=jax.ShapeDtypeStruct((2, NPAD), jnp.float32),
    mesh=_MESH,
    scratch_types=[
        pltpu.VMEM((DEG_ROWS, DCHK), jnp.int32),
        pltpu.VMEM((DCHK,), jnp.float32),
        pltpu.VMEM((NPAD // 16,), jnp.float32),
        pltpu.VMEM_SHARED((NPAD,), jnp.float32),
    ],
)
def _deg_kernel(dstd_hbm, out_hbm, di_v, ones_v, z_v, acc):
    c = lax.axis_index("c")
    s = lax.axis_index("s")
    for i in range(DCHK // 16):
        ones_v[pl.ds(i * 16, 16)] = jnp.full((16,), 1.0, jnp.float32)
    for i in range((NPAD // 16) // 16):
        z_v[pl.ds(i * 16, 16)] = jnp.zeros((16,), jnp.float32)
    strip = NPAD // 16
    pltpu.sync_copy(z_v, acc.at[pl.ds(s * strip, strip)])
    w = c * 16 + s
    pltpu.sync_copy(dstd_hbm.at[pl.ds(w * DEG_ROWS, DEG_ROWS)], di_v)
    plsc.subcore_barrier()

    def body(i, _):
        pltpu.sync_copy(ones_v, acc.at[di_v.at[i]], add=True)
        return ()

    lax.fori_loop(0, DEG_ROWS, body, ())
    plsc.subcore_barrier()
    pltpu.sync_copy(acc.at[pl.ds(s * strip, strip)],
                    out_hbm.at[c, pl.ds(s * strip, strip)])


@functools.partial(
    pl.kernel,
    out_type=jax.ShapeDtypeStruct((2 * NPAD, D), jnp.float32),
    mesh=_MESH,
    scratch_types=[
        pltpu.VMEM((G, CHK), jnp.int32),
        pltpu.VMEM((G, CHK), jnp.int32),
        pltpu.VMEM((CHK, D), jnp.float32),
        pltpu.VMEM((CHK, D), jnp.float32),
        pltpu.VMEM_SHARED((NPAD, D), jnp.float32),
        pltpu.SemaphoreType.DMA,
        pltpu.SemaphoreType.DMA,
    ],
)
def _prop_kernel(hs_hbm, src2p_hbm, dst2_hbm, zrows_hbm, out_hbm,
                 si_v, di_v, rows_a, rows_b, acc, sem_a, sem_b):
    c = lax.axis_index("c")
    s = lax.axis_index("s")
    row0 = s * ROWS_PER_TILE

    def group(g, _):
        pltpu.sync_copy(
            src2p_hbm.at[pl.ds(c * NROW2 + s * NCHNK + g * G, G)], si_v)
        pltpu.sync_copy(dst2_hbm.at[pl.ds(s * NCHNK + g * G, G)], di_v)

        def pair(i, _):
            j0 = 2 * i
            j1 = 2 * i + 1
            ha = pltpu.async_copy(hs_hbm.at[si_v.at[j0]], rows_a, sem_a)
            hb = pltpu.async_copy(hs_hbm.at[si_v.at[j1]], rows_b, sem_b)
            ha.wait()
            pltpu.sync_copy(rows_a, acc.at[di_v.at[j0]], add=True)
            hb.wait()
            pltpu.sync_copy(rows_b, acc.at[di_v.at[j1]], add=True)
            return ()

        lax.fori_loop(0, G // 2, pair, ())
        return ()

    for q in range(ROWS_PER_TILE // CHK):
        pltpu.sync_copy(zrows_hbm, acc.at[pl.ds(row0 + q * CHK, CHK)])
    plsc.subcore_barrier()
    lax.fori_loop(0, NCHNK // G, group, ())
    plsc.subcore_barrier()
    pltpu.sync_copy(acc.at[pl.ds(row0, ROWS_PER_TILE)],
                    out_hbm.at[pl.ds(c * NPAD + row0, ROWS_PER_TILE)])


_RB = 2000


def _mm_body(x_ref, w_ref, b_ref, degt_ref, hhp_ref, hs_ref, hi_ref):
    xb = x_ref[...]
    acc = jnp.dot(xb, w_ref[...], preferred_element_type=jnp.float32)
    acc = acc + b_ref[0:1, :]
    degb = degt_ref[...]
    deg = degb[:, 0:1] + degb[:, 1:2]
    dinv = jnp.where(deg > 0, lax.rsqrt(deg), 0.0)
    h_hp = acc[:, 0:D]
    h_lp = acc[:, D:2 * D]
    h_i = acc[:, 2 * D:3 * D]
    hhp_ref[...] = h_hp
    hs_ref[0] = dinv * h_hp
    hs_ref[1] = dinv * h_lp
    hi_ref[...] = jnp.maximum(h_i, 0.0)


def _mm_call(x, w_cat, b_cat, deg_t):
    return pl.pallas_call(
        _mm_body,
        grid=(N // _RB,),
        in_specs=[
            pl.BlockSpec((_RB, D), lambda i: (i, 0)),
            pl.BlockSpec((D, 3 * D), lambda i: (0, 0)),
            pl.BlockSpec((8, 3 * D), lambda i: (0, 0)),
            pl.BlockSpec((_RB, 2), lambda i: (i, 0)),
        ],
        out_specs=[
            pl.BlockSpec((_RB, D), lambda i: (i, 0)),
            pl.BlockSpec((2, _RB, D), lambda i: (0, i, 0)),
            pl.BlockSpec((_RB, D), lambda i: (i, 0)),
        ],
        out_shape=[
            jax.ShapeDtypeStruct((N, D), jnp.float32),
            jax.ShapeDtypeStruct((2, NPAD, D), jnp.float32),
            jax.ShapeDtypeStruct((N, D), jnp.float32),
        ],
    )(x, w_cat, b_cat, deg_t)


def _fin_body(hhp_ref, prop_ref, hi_ref, degt_ref, gw_ref,
              hhp0_ref, prop0_ref, hi0_ref, deg0_ref, out_ref):
    d0 = deg0_ref[0, 0] + deg0_ref[0, 1]
    dinv0 = jnp.where(d0 > 0, lax.rsqrt(d0), 0.0)
    php0 = prop0_ref[0, 0:1, :]
    plp0 = prop0_ref[1, 0:1, :]
    hhp0 = jnp.maximum(hhp0_ref[0:1, :] - dinv0 * php0, 0.0)
    hlp0 = jnp.maximum(dinv0 * plp0, 0.0)
    hi0 = hi0_ref[0:1, :]
    ga = jnp.sum(hhp0 * gw_ref[0:1, :]) + gw_ref[3, 0]
    gb = jnp.sum(hlp0 * gw_ref[1:2, :]) + gw_ref[3, 1]
    gc = jnp.sum(hi0 * gw_ref[2:3, :]) + gw_ref[3, 2]

    degb = degt_ref[...]
    deg = degb[:, 0:1] + degb[:, 1:2]
    dinv = jnp.where(deg > 0, lax.rsqrt(deg), 0.0)
    php = prop_ref[0]
    plp = prop_ref[1]
    h_hp = jnp.maximum(hhp_ref[...] - dinv * php, 0.0)
    h_lp = jnp.maximum(dinv * plp, 0.0)
    z = ga * h_hp + gb * h_lp + gc * hi_ref[...]
    m = jnp.max(z, axis=1, keepdims=True)
    zs = z - m
    out_ref[...] = zs - jnp.log(jnp.sum(jnp.exp(zs), axis=1, keepdims=True))


def _fin_call(h_hp, prop, h_i, deg_t, gw):
    return pl.pallas_call(
        _fin_body,
        grid=(N // _RB,),
        in_specs=[
            pl.BlockSpec((_RB, D), lambda i: (i, 0)),
            pl.BlockSpec((2, _RB, D), lambda i: (0, i, 0)),
            pl.BlockSpec((_RB, D), lambda i: (i, 0)),
            pl.BlockSpec((_RB, 2), lambda i: (i, 0)),
            pl.BlockSpec((8, D), lambda i: (0, 0)),
            pl.BlockSpec((8, D), lambda i: (0, 0)),
            pl.BlockSpec((2, 8, D), lambda i: (0, 0, 0)),
            pl.BlockSpec((8, D), lambda i: (0, 0)),
            pl.BlockSpec((8, 2), lambda i: (0, 0)),
        ],
        out_specs=pl.BlockSpec((_RB, D), lambda i: (i, 0)),
        out_shape=jax.ShapeDtypeStruct((N, D), jnp.float32),
    )(h_hp, prop, h_i, deg_t, gw, h_hp, prop, h_i, deg_t)


def kernel(x, edge_index, W_hp, b_hp, W_lp, b_lp, W_i, b_i,
           w_h, bh, w_l, bl, w_i, bi):
    src = edge_index[0]
    dst = edge_index[1]

    pad = NE_PAD - NE
    src_pad = jnp.concatenate([src, jnp.zeros((pad,), jnp.int32)])
    dst_pad = jnp.concatenate([dst, jnp.full((pad,), N, jnp.int32)])
    src2 = src_pad.reshape(NROW2, CHK)
    dst2 = dst_pad.reshape(NROW2, CHK)

    deg2 = _deg_kernel(dst2)
    deg_t = jnp.transpose(deg2)[:N]

    w_cat = jnp.concatenate([W_hp, W_lp, W_i], axis=1)
    b_cat = jnp.zeros((8, 3 * D), jnp.float32).at[0].set(
        jnp.concatenate([b_hp, b_lp, b_i]))

    h_hp, hs, h_i = _mm_call(x, w_cat, b_cat, deg_t)

    zrows = jnp.zeros((CHK, D), jnp.float32)

    hs_flat = hs.reshape(2 * NPAD, D)
    src2p = (src2[None, :, :]
             + (jnp.arange(2, dtype=jnp.int32) * NPAD)[:, None, None]
             ).reshape(2 * NROW2, CHK)

    prop = _prop_kernel(hs_flat, src2p, dst2, zrows).reshape(2, NPAD, D)

    gw = (jnp.zeros((8, D), jnp.float32)
          .at[0].set(w_h[:, 0]).at[1].set(w_l[:, 0]).at[2].set(w_i[:, 0])
          .at[3, 0].set(bh[0]).at[3, 1].set(bl[0]).at[3, 2].set(bi[0]))

    return _fin_call(h_hp, prop, h_i, deg_t, gw)


# speedup vs baseline: 1.0529x; 1.0065x over previous
import functools

import jax
import jax.numpy as jnp
from jax import lax
from jax.experimental import pallas as pl
from jax.experimental.pallas import tpu as pltpu
from jax.experimental.pallas import tpu_sc as plsc

N = 10000
D = 128
DH = D // 2
NE = 320000
NPAD = 10240
CHK = 128
NCHNK = 160
EPT = NCHNK * CHK
NE_PAD = EPT * 16
NROW2 = NE_PAD // CHK
G = 16
DCHK = 128
DEG_ROWS = NE_PAD // DCHK // 32
ROWS_PER_TILE = NPAD // 16

_MESH = plsc.VectorSubcoreMesh(core_axis_name="c", subcore_axis_name="s")


@functools.partial(
    pl.kernel,
    out_type=jax.ShapeDtypeStruct((2, NPAD), jnp.float32),
    mesh=_MESH,
    scratch_types=[
        pltpu.VMEM((DEG_ROWS, DCHK), jnp.int32),
        pltpu.VMEM((DCHK,), jnp.float32),
        pltpu.VMEM((NPAD // 16,), jnp.float32),
        pltpu.VMEM_SHARED((NPAD,), jnp.float32),
    ],
)
def _deg_kernel(dstd_hbm, out_hbm, di_v, ones_v, z_v, acc):
    c = lax.axis_index("c")
    s = lax.axis_index("s")
    for i in range(DCHK // 16):
        ones_v[pl.ds(i * 16, 16)] = jnp.full((16,), 1.0, jnp.float32)
    for i in range((NPAD // 16) // 16):
        z_v[pl.ds(i * 16, 16)] = jnp.zeros((16,), jnp.float32)
    strip = NPAD // 16
    pltpu.sync_copy(z_v, acc.at[pl.ds(s * strip, strip)])
    w = c * 16 + s
    pltpu.sync_copy(dstd_hbm.at[pl.ds(w * DEG_ROWS, DEG_ROWS)], di_v)
    plsc.subcore_barrier()

    def body(i, _):
        pltpu.sync_copy(ones_v, acc.at[di_v.at[i]], add=True)
        return ()

    lax.fori_loop(0, DEG_ROWS, body, ())
    plsc.subcore_barrier()
    pltpu.sync_copy(acc.at[pl.ds(s * strip, strip)],
                    out_hbm.at[c, pl.ds(s * strip, strip)])


@functools.partial(
    pl.kernel,
    out_type=jax.ShapeDtypeStruct((2 * NPAD, D), jnp.float32),
    mesh=_MESH,
    scratch_types=[
        pltpu.VMEM((G, CHK), jnp.int32),
        pltpu.VMEM((G, CHK), jnp.int32),
        pltpu.VMEM((CHK, D), jnp.float32),
        pltpu.VMEM((CHK, D), jnp.float32),
        pltpu.VMEM_SHARED((NPAD, D), jnp.float32),
        pltpu.SemaphoreType.DMA,
        pltpu.SemaphoreType.DMA,
        pltpu.SemaphoreType.DMA,
        pltpu.SemaphoreType.DMA,
    ],
)
def _prop_kernel(hs_hbm, src2p_hbm, dst2_hbm, zrows_hbm, out_hbm,
                 si_v, di_v, rows_a, rows_b, acc, sem_a, sem_b,
                 sem_sa, sem_sb):
    c = lax.axis_index("c")
    s = lax.axis_index("s")
    row0 = s * ROWS_PER_TILE

    def group(g, _):
        pltpu.sync_copy(
            src2p_hbm.at[pl.ds(c * NROW2 + s * NCHNK + g * G, G)], si_v)
        pltpu.sync_copy(dst2_hbm.at[pl.ds(s * NCHNK + g * G, G)], di_v)

        def pair(i, _):
            j0 = 2 * i
            j1 = 2 * i + 1
            ha = pltpu.async_copy(hs_hbm.at[si_v.at[j0]], rows_a, sem_a)
            hb = pltpu.async_copy(hs_hbm.at[si_v.at[j1]], rows_b, sem_b)
            ha.wait()
            sa = pltpu.async_copy(rows_a, acc.at[di_v.at[j0]], sem_sa,
                                  add=True)
            hb.wait()
            sb = pltpu.async_copy(rows_b, acc.at[di_v.at[j1]], sem_sb,
                                  add=True)
            sa.wait()
            sb.wait()
            return ()

        lax.fori_loop(0, G // 2, pair, ())
        return ()

    for q in range(ROWS_PER_TILE // CHK):
        pltpu.sync_copy(zrows_hbm, acc.at[pl.ds(row0 + q * CHK, CHK)])
    plsc.subcore_barrier()
    lax.fori_loop(0, NCHNK // G, group, ())
    plsc.subcore_barrier()
    pltpu.sync_copy(acc.at[pl.ds(row0, ROWS_PER_TILE)],
                    out_hbm.at[pl.ds(c * NPAD + row0, ROWS_PER_TILE)])


_RB = 2000


def _mm_body(x_ref, w_ref, b_ref, degt_ref, hhp_ref, hs_ref, hi_ref):
    xb = x_ref[...]
    acc = jnp.dot(xb, w_ref[...], preferred_element_type=jnp.float32)
    acc = acc + b_ref[0:1, :]
    degb = degt_ref[...]
    deg = degb[:, 0:1] + degb[:, 1:2]
    dinv = jnp.where(deg > 0, lax.rsqrt(deg), 0.0)
    h_hp = acc[:, 0:D]
    h_lp = acc[:, D:2 * D]
    h_i = acc[:, 2 * D:3 * D]
    hhp_ref[...] = h_hp
    hs_ref[0] = dinv * h_hp
    hs_ref[1] = dinv * h_lp
    hi_ref[...] = jnp.maximum(h_i, 0.0)


def _mm_call(x, w_cat, b_cat, deg_t):
    return pl.pallas_call(
        _mm_body,
        grid=(N // _RB,),
        in_specs=[
            pl.BlockSpec((_RB, D), lambda i: (i, 0)),
            pl.BlockSpec((D, 3 * D), lambda i: (0, 0)),
            pl.BlockSpec((8, 3 * D), lambda i: (0, 0)),
            pl.BlockSpec((_RB, 2), lambda i: (i, 0)),
        ],
        out_specs=[
            pl.BlockSpec((_RB, D), lambda i: (i, 0)),
            pl.BlockSpec((2, _RB, D), lambda i: (0, i, 0)),
            pl.BlockSpec((_RB, D), lambda i: (i, 0)),
        ],
        out_shape=[
            jax.ShapeDtypeStruct((N, D), jnp.float32),
            jax.ShapeDtypeStruct((2, NPAD, D), jnp.float32),
            jax.ShapeDtypeStruct((N, D), jnp.float32),
        ],
    )(x, w_cat, b_cat, deg_t)


def _fin_body(hhp_ref, prop_ref, hi_ref, degt_ref, gw_ref,
              hhp0_ref, prop0_ref, hi0_ref, deg0_ref, out_ref):
    d0 = deg0_ref[0, 0] + deg0_ref[0, 1]
    dinv0 = jnp.where(d0 > 0, lax.rsqrt(d0), 0.0)
    php0 = prop0_ref[0, 0:1, :]
    plp0 = prop0_ref[1, 0:1, :]
    hhp0 = jnp.maximum(hhp0_ref[0:1, :] - dinv0 * php0, 0.0)
    hlp0 = jnp.maximum(dinv0 * plp0, 0.0)
    hi0 = hi0_ref[0:1, :]
    ga = jnp.sum(hhp0 * gw_ref[0:1, :]) + gw_ref[3, 0]
    gb = jnp.sum(hlp0 * gw_ref[1:2, :]) + gw_ref[3, 1]
    gc = jnp.sum(hi0 * gw_ref[2:3, :]) + gw_ref[3, 2]

    degb = degt_ref[...]
    deg = degb[:, 0:1] + degb[:, 1:2]
    dinv = jnp.where(deg > 0, lax.rsqrt(deg), 0.0)
    php = prop_ref[0]
    plp = prop_ref[1]
    h_hp = jnp.maximum(hhp_ref[...] - dinv * php, 0.0)
    h_lp = jnp.maximum(dinv * plp, 0.0)
    z = ga * h_hp + gb * h_lp + gc * hi_ref[...]
    m = jnp.max(z, axis=1, keepdims=True)
    zs = z - m
    out_ref[...] = zs - jnp.log(jnp.sum(jnp.exp(zs), axis=1, keepdims=True))


def _fin_call(h_hp, prop, h_i, deg_t, gw):
    return pl.pallas_call(
        _fin_body,
        grid=(N // _RB,),
        in_specs=[
            pl.BlockSpec((_RB, D), lambda i: (i, 0)),
            pl.BlockSpec((2, _RB, D), lambda i: (0, i, 0)),
            pl.BlockSpec((_RB, D), lambda i: (i, 0)),
            pl.BlockSpec((_RB, 2), lambda i: (i, 0)),
            pl.BlockSpec((8, D), lambda i: (0, 0)),
            pl.BlockSpec((8, D), lambda i: (0, 0)),
            pl.BlockSpec((2, 8, D), lambda i: (0, 0, 0)),
            pl.BlockSpec((8, D), lambda i: (0, 0)),
            pl.BlockSpec((8, 2), lambda i: (0, 0)),
        ],
        out_specs=pl.BlockSpec((_RB, D), lambda i: (i, 0)),
        out_shape=jax.ShapeDtypeStruct((N, D), jnp.float32),
    )(h_hp, prop, h_i, deg_t, gw, h_hp, prop, h_i, deg_t)


def kernel(x, edge_index, W_hp, b_hp, W_lp, b_lp, W_i, b_i,
           w_h, bh, w_l, bl, w_i, bi):
    src = edge_index[0]
    dst = edge_index[1]

    pad = NE_PAD - NE
    src_pad = jnp.concatenate([src, jnp.zeros((pad,), jnp.int32)])
    dst_pad = jnp.concatenate([dst, jnp.full((pad,), N, jnp.int32)])
    src2 = src_pad.reshape(NROW2, CHK)
    dst2 = dst_pad.reshape(NROW2, CHK)

    deg2 = _deg_kernel(dst2)
    deg_t = jnp.transpose(deg2)[:N]

    w_cat = jnp.concatenate([W_hp, W_lp, W_i], axis=1)
    b_cat = jnp.zeros((8, 3 * D), jnp.float32).at[0].set(
        jnp.concatenate([b_hp, b_lp, b_i]))

    h_hp, hs, h_i = _mm_call(x, w_cat, b_cat, deg_t)

    zrows = jnp.zeros((CHK, D), jnp.float32)

    hs_flat = hs.reshape(2 * NPAD, D)
    src2p = (src2[None, :, :]
             + (jnp.arange(2, dtype=jnp.int32) * NPAD)[:, None, None]
             ).reshape(2 * NROW2, CHK)

    prop = _prop_kernel(hs_flat, src2p, dst2, zrows).reshape(2, NPAD, D)

    gw = (jnp.zeros((8, D), jnp.float32)
          .at[0].set(w_h[:, 0]).at[1].set(w_l[:, 0]).at[2].set(w_i[:, 0])
          .at[3, 0].set(bh[0]).at[3, 1].set(bl[0]).at[3, 2].set(bi[0]))

    return _fin_call(h_hp, prop, h_i, deg_t, gw)
